# baseline (speedup 1.0000x reference)
import jax
import jax.numpy as jnp
from jax.experimental import pallas as pl

B = 262144
NUM_TILES = 64
SIG_DIM = 16
PACK = 8
RQ = B // PACK
RBQ = 1024


def _body(sq_ref, w_ref, scores_ref, idx_ref):
    sq = sq_ref[...]
    w = w_ref[...]
    sc = jax.lax.dot_general(
        sq, w, (((1,), (0,)), ((), ())),
        preferred_element_type=jnp.float32)
    scores_ref[...] = sc
    st = sc.T
    g = st.reshape(PACK, NUM_TILES, RBQ)
    mx = jnp.max(g, axis=1, keepdims=True)
    iota = jax.lax.broadcasted_iota(jnp.int32, g.shape, 1)
    idx = jnp.min(jnp.where(g == mx, iota, NUM_TILES), axis=1)
    idx_ref[...] = idx.T


def kernel(sig, tile_signatures):
    sq = sig.reshape(RQ, PACK * SIG_DIM)
    wt = tile_signatures.T
    zero = jnp.zeros_like(wt)
    wbig = jnp.concatenate(
        [jnp.concatenate([wt if i == j else zero for j in range(PACK)], axis=1)
         for i in range(PACK)], axis=0)
    scores, idx = pl.pallas_call(
        _body,
        grid=(RQ // RBQ,),
        in_specs=[
            pl.BlockSpec((RBQ, PACK * SIG_DIM), lambda i: (i, 0)),
            pl.BlockSpec((PACK * SIG_DIM, PACK * NUM_TILES), lambda i: (0, 0)),
        ],
        out_specs=[
            pl.BlockSpec((RBQ, PACK * NUM_TILES), lambda i: (i, 0)),
            pl.BlockSpec((RBQ, PACK), lambda i: (i, 0)),
        ],
        out_shape=[
            jax.ShapeDtypeStruct((RQ, PACK * NUM_TILES), jnp.float32),
            jax.ShapeDtypeStruct((RQ, PACK), jnp.int32),
        ],
    )(sq, wbig)
    return scores.reshape(B, NUM_TILES), idx.reshape(B)


# speedup vs baseline: 7.3080x; 7.3080x over previous
import jax
import jax.numpy as jnp
from jax.experimental import pallas as pl

B = 262144
NUM_TILES = 64
SIG_DIM = 16
RB = 8192


def _body(sigt_ref, tsig_ref, scorest_ref, idx_ref):
    st = sigt_ref[...]
    t = tsig_ref[...]
    sc = jax.lax.dot_general(
        t, st, (((1,), (0,)), ((), ())),
        preferred_element_type=jnp.float32)
    scorest_ref[...] = sc
    mx = jnp.max(sc, axis=0, keepdims=True)
    iota = jax.lax.broadcasted_iota(jnp.int32, sc.shape, 0)
    idx_ref[...] = jnp.min(jnp.where(sc == mx, iota, NUM_TILES), axis=0)


def kernel(sig, tile_signatures):
    sigt = sig.T
    scorest, idx = pl.pallas_call(
        _body,
        grid=(B // RB,),
        in_specs=[
            pl.BlockSpec((SIG_DIM, RB), lambda i: (0, i)),
            pl.BlockSpec((NUM_TILES, SIG_DIM), lambda i: (0, 0)),
        ],
        out_specs=[
            pl.BlockSpec((NUM_TILES, RB), lambda i: (0, i)),
            pl.BlockSpec((RB,), lambda i: (i,)),
        ],
        out_shape=[
            jax.ShapeDtypeStruct((NUM_TILES, B), jnp.float32),
            jax.ShapeDtypeStruct((B,), jnp.int32),
        ],
    )(sigt, tile_signatures)
    return scorest.T, idx
